# baseline (speedup 1.0000x reference)
import dataclasses

import jax
import jax.numpy as jnp
from jax import lax
from jax.experimental import pallas as pl
from jax.experimental.pallas import tpu as pltpu
from jax.experimental.pallas import tpu_sc as plsc

N = 10000
E = 320000
D = 128
HD = D // 2

NC = 2
NS = 16
NW = NC * NS
BLK = 128
NB = 80
EPW = NB * BLK
EP = NW * EPW
NP = 10112
RPS = NP // NS


def _mesh():
    return plsc.VectorSubcoreMesh(core_axis_name="c", subcore_axis_name="s")


def _deg_body(dst_hbm, degp_hbm, dstv, hist):
    c = lax.axis_index("c")
    s = lax.axis_index("s")
    wid = s * NC + c
    ones16 = jnp.ones((16,), jnp.float32)

    @pl.loop(0, NP // 16)
    def _(i):
        hist[pl.ds(i * 16, 16)] = jnp.zeros((16,), jnp.float32)

    pltpu.sync_copy(dst_hbm.at[wid], dstv)

    @pl.loop(0, NB)
    def _(j):
        @pl.loop(0, BLK // 16)
        def _(k):
            idx = dstv[j, pl.ds(k * 16, 16)]
            plsc.addupdate_scatter(hist, [idx], ones16)

    pltpu.sync_copy(hist, degp_hbm.at[wid, 0])


def _agg_body(src_hbm, dst_hbm, hs_hbm, zeros_hbm, agg_hbm, srcv, dstv,
              r0, r1, aggs, sg, ss):
    c = lax.axis_index("c")
    s = lax.axis_index("s")
    wid = s * NC + c

    pltpu.sync_copy(src_hbm.at[wid], srcv)
    pltpu.sync_copy(dst_hbm.at[wid], dstv)
    pltpu.sync_copy(zeros_hbm.at[pl.ds(s * RPS, RPS)],
                    aggs.at[pl.ds(s * RPS, RPS)])

    plsc.subcore_barrier()

    @pl.loop(0, NB)
    def _(j):
        pltpu.sync_copy(hs_hbm.at[srcv.at[j]], r0)
        pltpu.sync_copy(r0, aggs.at[dstv.at[j]], add=True)

    plsc.subcore_barrier()
    pltpu.sync_copy(aggs.at[pl.ds(s * RPS, RPS)],
                    agg_hbm.at[c, pl.ds(s * RPS, RPS)])


def _compiler_params():
    cp = pltpu.CompilerParams()
    if "needs_layout_passes" in pltpu.CompilerParams.__dataclass_fields__:
        cp = dataclasses.replace(cp, needs_layout_passes=False)
    return cp


def _sc_deg(dst_r):
    k = pl.kernel(
        _deg_body,
        out_type=jax.ShapeDtypeStruct((NW, 1, NP), jnp.float32),
        mesh=_mesh(),
        compiler_params=_compiler_params(),
        scratch_types=[
            pltpu.VMEM((NB, BLK), jnp.int32),
            pltpu.VMEM((NP,), jnp.float32),
        ],
    )
    return k(dst_r)


def _degsum_body(degp_ref, deg_ref):
    p = degp_ref[...].reshape(NW, NP)
    ones = jnp.ones((NW, 16), jnp.float32)
    deg_ref[...] = lax.dot_general(p, ones, (((0,), (0,)), ((), ())),
                                   preferred_element_type=jnp.float32)


def _tc_degsum(degp):
    return pl.pallas_call(
        _degsum_body,
        grid=(1,),
        in_specs=[pl.BlockSpec((NW, 1, NP), lambda i: (0, 0, 0))],
        out_specs=pl.BlockSpec((NP, 16), lambda i: (0, 0)),
        out_shape=jax.ShapeDtypeStruct((NP, 16), jnp.float32),
    )(degp)


def _sc_agg(src_r, dst_r, hs2):
    zeros = jnp.zeros((NP, D), jnp.float32)
    k = pl.kernel(
        _agg_body,
        out_type=jax.ShapeDtypeStruct((NC, NP, D), jnp.float32),
        mesh=_mesh(),
        scratch_types=[
            pltpu.VMEM((NB, BLK), jnp.int32),
            pltpu.VMEM((NB, BLK), jnp.int32),
            pltpu.VMEM((BLK, D), jnp.float32),
            pltpu.VMEM((BLK, D), jnp.float32),
            pltpu.VMEM_SHARED((NP, D), jnp.float32),
            pltpu.SemaphoreType.DMA,
            pltpu.SemaphoreType.DMA,
        ],
    )
    return k(src_r, dst_r, hs2, zeros)


_RB = 1000
_NRB = N // _RB


def _hs_body(x_ref, w_ref, deg_ref, hs_ref):
    degsum = deg_ref[:, 0] + 1.0
    dinv = lax.rsqrt(degsum)
    h = lax.dot_general(x_ref[...], w_ref[...],
                        (((1,), (1,)), ((), ())),
                        preferred_element_type=jnp.float32)
    hs_ref[...] = h * dinv[:, None]


def _tc_hs(x, W, deg):
    return pl.pallas_call(
        _hs_body,
        grid=(_NRB,),
        in_specs=[
            pl.BlockSpec((_RB, D), lambda i: (i, 0)),
            pl.BlockSpec((D, D), lambda i: (0, 0)),
            pl.BlockSpec((_RB, 16), lambda i: (i, 0)),
        ],
        out_specs=pl.BlockSpec((_RB, D), lambda i: (i, 0)),
        out_shape=jax.ShapeDtypeStruct((N, D), jnp.float32),
    )(x, W, deg)


def _pre_block(agg_ref, hs_ref, deg_ref, b_ref):
    degsum = deg_ref[:, 0] + 1.0
    dinv = lax.rsqrt(degsum)
    agg = agg_ref[0] + agg_ref[1]
    return dinv[:, None] * (agg + hs_ref[...]) + b_ref[...][None, :]


def _stats_body(agg_ref, hs_ref, deg_ref, b_ref, stats_ref, acc_ref):
    t = pl.program_id(0)
    pre = _pre_block(agg_ref, hs_ref, deg_ref, b_ref)

    @pl.when(t == 0)
    def _():
        acc_ref[...] = jnp.zeros_like(acc_ref)

    acc_ref[0, :] += jnp.sum(pre, axis=0)
    acc_ref[1, :] += jnp.sum(pre * pre, axis=0)

    @pl.when(t == _NRB - 1)
    def _():
        mean = acc_ref[0, :] * (1.0 / N)
        var = acc_ref[1, :] * (1.0 / N) - mean * mean
        stats_ref[0, :] = mean
        stats_ref[1, :] = lax.rsqrt(var + 1e-5)


def _norm_body(agg_ref, hs_ref, deg_ref, b_ref, stats_ref, g_ref, be_ref,
               out_ref):
    pre = _pre_block(agg_ref, hs_ref, deg_ref, b_ref)
    y = (pre - stats_ref[0, :][None, :]) * stats_ref[1, :][None, :]
    out_ref[...] = jnp.maximum(
        y * g_ref[...][None, :] + be_ref[...][None, :], 0.0)


def _tc_bn(agg, hs2, deg, b, gamma, beta):
    common = [
        pl.BlockSpec((NC, _RB, D), lambda t: (0, t, 0)),
        pl.BlockSpec((_RB, D), lambda t: (t, 0)),
        pl.BlockSpec((_RB, 16), lambda t: (t, 0)),
        pl.BlockSpec((D,), lambda t: (0,)),
    ]
    stats = pl.pallas_call(
        _stats_body,
        grid=(_NRB,),
        in_specs=common,
        out_specs=pl.BlockSpec((2, D), lambda t: (0, 0)),
        out_shape=jax.ShapeDtypeStruct((2, D), jnp.float32),
        scratch_shapes=[pltpu.VMEM((2, D), jnp.float32)],
    )(agg, hs2, deg, b)
    return pl.pallas_call(
        _norm_body,
        grid=(_NRB,),
        in_specs=common + [
            pl.BlockSpec((2, D), lambda t: (0, 0)),
            pl.BlockSpec((D,), lambda t: (0,)),
            pl.BlockSpec((D,), lambda t: (0,)),
        ],
        out_specs=pl.BlockSpec((_RB, D), lambda t: (t, 0)),
        out_shape=jax.ShapeDtypeStruct((N, D), jnp.float32),
    )(agg, hs2, deg, b, stats, gamma, beta)


def kernel(x, edge_index, W, b, gamma, beta):
    src = edge_index[0].astype(jnp.int32)
    dst = edge_index[1].astype(jnp.int32)
    ppw = NB * BLK - E // NW
    pad_src = jnp.zeros((NW, ppw), jnp.int32)
    pad_dst = jnp.broadcast_to(N + (jnp.arange(ppw, dtype=jnp.int32)
                                    % (NP - N)), (NW, ppw))
    src = jnp.concatenate([src.reshape(NW, E // NW), pad_src], axis=1)
    dst = jnp.concatenate([dst.reshape(NW, E // NW), pad_dst], axis=1)
    src = src.reshape(NW, NB, BLK)
    dst = dst.reshape(NW, NB, BLK)
    deg = _tc_degsum(_sc_deg(dst))
    hs2 = _tc_hs(x, W, deg)
    agg = _sc_agg(src, dst, hs2)
    return _tc_bn(agg, hs2, deg, b, gamma, beta)


# speedup vs baseline: 1.8645x; 1.8645x over previous
import dataclasses

import jax
import jax.numpy as jnp
from jax import lax
from jax.experimental import pallas as pl
from jax.experimental.pallas import tpu as pltpu
from jax.experimental.pallas import tpu_sc as plsc

N = 10000
E = 320000
D = 128
HD = D // 2

NC = 2
NS = 16
NW = NC * NS
BLK = 80
NB = 125
EPW = NB * BLK
EP = NW * EPW
NP = 10112
RPS = NP // NS


def _mesh():
    return plsc.VectorSubcoreMesh(core_axis_name="c", subcore_axis_name="s")


def _deg_body(dst_hbm, degp_hbm, dstv, hist):
    c = lax.axis_index("c")
    s = lax.axis_index("s")
    wid = s * NC + c
    ones16 = jnp.ones((16,), jnp.float32)

    @pl.loop(0, NP // 16)
    def _(i):
        hist[pl.ds(i * 16, 16)] = jnp.zeros((16,), jnp.float32)

    pltpu.sync_copy(dst_hbm.at[wid], dstv)

    @pl.loop(0, NB)
    def _(j):
        @pl.loop(0, BLK // 16)
        def _(k):
            idx = dstv[j, pl.ds(k * 16, 16)]
            plsc.addupdate_scatter(hist, [idx], ones16)

    pltpu.sync_copy(hist, degp_hbm.at[wid, 0])


def _agg_body(src_hbm, dst_hbm, hs_hbm, zeros_hbm, agg_hbm, srcv, dstv,
              rows, aggs):
    c = lax.axis_index("c")
    s = lax.axis_index("s")
    wid = s * NC + c

    pltpu.sync_copy(src_hbm.at[wid], srcv)
    pltpu.sync_copy(dst_hbm.at[wid], dstv)
    pltpu.sync_copy(zeros_hbm.at[pl.ds(s * RPS, RPS)],
                    aggs.at[pl.ds(s * RPS, RPS)])

    plsc.subcore_barrier()

    @pl.loop(0, NB)
    def _(j):
        pltpu.sync_copy(hs_hbm.at[srcv.at[j]], rows)
        pltpu.sync_copy(rows, aggs.at[dstv.at[j]], add=True)

    plsc.subcore_barrier()
    pltpu.sync_copy(aggs.at[pl.ds(s * RPS, RPS)],
                    agg_hbm.at[c, pl.ds(s * RPS, RPS)])


def _compiler_params():
    cp = pltpu.CompilerParams()
    if "needs_layout_passes" in pltpu.CompilerParams.__dataclass_fields__:
        cp = dataclasses.replace(cp, needs_layout_passes=False)
    return cp


def _sc_deg(dst_r):
    k = pl.kernel(
        _deg_body,
        out_type=jax.ShapeDtypeStruct((NW, 1, NP), jnp.float32),
        mesh=_mesh(),
        compiler_params=_compiler_params(),
        scratch_types=[
            pltpu.VMEM((NB, BLK), jnp.int32),
            pltpu.VMEM((NP,), jnp.float32),
        ],
    )
    return k(dst_r)


def _degsum_body(degp_ref, deg_ref):
    p = degp_ref[...].reshape(NW, NP)
    ones = jnp.ones((NW, 16), jnp.float32)
    deg_ref[...] = lax.dot_general(p, ones, (((0,), (0,)), ((), ())),
                                   preferred_element_type=jnp.float32)


def _tc_degsum(degp):
    return pl.pallas_call(
        _degsum_body,
        grid=(1,),
        in_specs=[pl.BlockSpec((NW, 1, NP), lambda i: (0, 0, 0))],
        out_specs=pl.BlockSpec((NP, 16), lambda i: (0, 0)),
        out_shape=jax.ShapeDtypeStruct((NP, 16), jnp.float32),
    )(degp)


def _sc_agg(src_r, dst_r, hs2):
    zeros = jnp.zeros((NP, D), jnp.float32)
    k = pl.kernel(
        _agg_body,
        out_type=jax.ShapeDtypeStruct((NC, NP, D), jnp.float32),
        mesh=_mesh(),
        scratch_types=[
            pltpu.VMEM((NB, BLK), jnp.int32),
            pltpu.VMEM((NB, BLK), jnp.int32),
            pltpu.VMEM((BLK, D), jnp.float32),
            pltpu.VMEM_SHARED((NP, D), jnp.float32),
        ],
    )
    return k(src_r, dst_r, hs2, zeros)


_RB = 1000
_NRB = N // _RB


def _hs_body(x_ref, w_ref, deg_ref, hs_ref):
    degsum = deg_ref[:, 0] + 1.0
    dinv = lax.rsqrt(degsum)
    h = lax.dot_general(x_ref[...], w_ref[...],
                        (((1,), (1,)), ((), ())),
                        preferred_element_type=jnp.float32)
    hs_ref[...] = h * dinv[:, None]


def _tc_hs(x, W, deg):
    return pl.pallas_call(
        _hs_body,
        grid=(_NRB,),
        in_specs=[
            pl.BlockSpec((_RB, D), lambda i: (i, 0)),
            pl.BlockSpec((D, D), lambda i: (0, 0)),
            pl.BlockSpec((_RB, 16), lambda i: (i, 0)),
        ],
        out_specs=pl.BlockSpec((_RB, D), lambda i: (i, 0)),
        out_shape=jax.ShapeDtypeStruct((N, D), jnp.float32),
    )(x, W, deg)


def _pre_block(agg_ref, hs_ref, deg_ref, b_ref):
    degsum = deg_ref[:, 0] + 1.0
    dinv = lax.rsqrt(degsum)
    agg = agg_ref[0] + agg_ref[1]
    return dinv[:, None] * (agg + hs_ref[...]) + b_ref[...][None, :]


def _stats_body(agg_ref, hs_ref, deg_ref, b_ref, stats_ref, acc_ref):
    t = pl.program_id(0)
    pre = _pre_block(agg_ref, hs_ref, deg_ref, b_ref)

    @pl.when(t == 0)
    def _():
        acc_ref[...] = jnp.zeros_like(acc_ref)

    acc_ref[0, :] += jnp.sum(pre, axis=0)
    acc_ref[1, :] += jnp.sum(pre * pre, axis=0)

    @pl.when(t == _NRB - 1)
    def _():
        mean = acc_ref[0, :] * (1.0 / N)
        var = acc_ref[1, :] * (1.0 / N) - mean * mean
        stats_ref[0, :] = mean
        stats_ref[1, :] = lax.rsqrt(var + 1e-5)


def _norm_body(agg_ref, hs_ref, deg_ref, b_ref, stats_ref, g_ref, be_ref,
               out_ref):
    pre = _pre_block(agg_ref, hs_ref, deg_ref, b_ref)
    y = (pre - stats_ref[0, :][None, :]) * stats_ref[1, :][None, :]
    out_ref[...] = jnp.maximum(
        y * g_ref[...][None, :] + be_ref[...][None, :], 0.0)


def _tc_bn(agg, hs2, deg, b, gamma, beta):
    common = [
        pl.BlockSpec((NC, _RB, D), lambda t: (0, t, 0)),
        pl.BlockSpec((_RB, D), lambda t: (t, 0)),
        pl.BlockSpec((_RB, 16), lambda t: (t, 0)),
        pl.BlockSpec((D,), lambda t: (0,)),
    ]
    stats = pl.pallas_call(
        _stats_body,
        grid=(_NRB,),
        in_specs=common,
        out_specs=pl.BlockSpec((2, D), lambda t: (0, 0)),
        out_shape=jax.ShapeDtypeStruct((2, D), jnp.float32),
        scratch_shapes=[pltpu.VMEM((2, D), jnp.float32)],
    )(agg, hs2, deg, b)
    return pl.pallas_call(
        _norm_body,
        grid=(_NRB,),
        in_specs=common + [
            pl.BlockSpec((2, D), lambda t: (0, 0)),
            pl.BlockSpec((D,), lambda t: (0,)),
            pl.BlockSpec((D,), lambda t: (0,)),
        ],
        out_specs=pl.BlockSpec((_RB, D), lambda t: (t, 0)),
        out_shape=jax.ShapeDtypeStruct((N, D), jnp.float32),
    )(agg, hs2, deg, b, stats, gamma, beta)


def kernel(x, edge_index, W, b, gamma, beta):
    src = edge_index[0].astype(jnp.int32).reshape(NW, NB, BLK)
    dst = edge_index[1].astype(jnp.int32).reshape(NW, NB, BLK)
    deg = _tc_degsum(_sc_deg(dst))
    hs2 = _tc_hs(x, W, deg)
    agg = _sc_agg(src, dst, hs2)
    return _tc_bn(agg, hs2, deg, b, gamma, beta)
